# baseline (speedup 1.0000x reference)
import functools

import jax
import jax.numpy as jnp
from jax import lax
from jax.experimental import pallas as pl
from jax.experimental.pallas import tpu as pltpu
from jax.experimental.pallas import tpu_sc as plsc

_NC = 2
_NS = 16
_NW = _NC * _NS
_CH = 128


def _gauss_body(e_total, p_ref, mut_ref, sgt_ref, g_ref):
    p = p_ref[...]
    a = -0.5 / (sgt_ref[...] ** 2 + 1e-14)
    mut = mut_ref[...]
    be, d = p.shape
    f = mut.shape[1]
    acc = jnp.zeros((be, f), jnp.float32)
    for dd in range(d):
        dif = p[:, dd:dd + 1] - mut[dd:dd + 1, :]
        acc = acc + a[dd:dd + 1, :] * (dif * dif)
    base = pl.program_id(0) * be
    eid = base + lax.broadcasted_iota(jnp.int32, (be, 1), 0)
    mask = (eid < e_total).astype(jnp.float32)
    g_ref[...] = jnp.exp(acc) * mask


def _gaussian(pseudo_pad, mut, sgt, e_total, be=2048):
    epad, d = pseudo_pad.shape
    f = mut.shape[1]
    grid = epad // be
    return pl.pallas_call(
        functools.partial(_gauss_body, e_total),
        grid=(grid,),
        in_specs=[
            pl.BlockSpec((be, d), lambda i: (i, 0)),
            pl.BlockSpec((d, f), lambda i: (0, 0)),
            pl.BlockSpec((d, f), lambda i: (0, 0)),
        ],
        out_specs=pl.BlockSpec((be, f), lambda i: (i, 0)),
        out_shape=jax.ShapeDtypeStruct((epad, f), jnp.float32),
    )(pseudo_pad, mut, sgt)


def _make_sc_scatter(n, f, epad):
    ew = epad // _NW
    n_chunks = ew // _CH
    rows_per_tile = n // _NS
    assert rows_per_tile * _NS == n and rows_per_tile % _CH == 0
    assert n_chunks % 4 == 0
    mesh = plsc.VectorSubcoreMesh(core_axis_name="c", subcore_axis_name="s",
                                  num_cores=_NC, num_subcores=_NS)

    @functools.partial(
        pl.kernel,
        out_type=jax.ShapeDtypeStruct((_NC, n, f), jnp.float32),
        mesh=mesh,
        scratch_types=[
            pltpu.VMEM((n_chunks, _CH), jnp.int32),
            pltpu.VMEM((n_chunks, _CH), jnp.int32),
            pltpu.VMEM((_CH, f), jnp.float32),
            pltpu.VMEM((_CH, f), jnp.float32),
            pltpu.VMEM_SHARED((n, f), jnp.float32),
            pltpu.SemaphoreType.DMA,
            pltpu.SemaphoreType.DMA,
        ],
    )
    def k(x_hbm, col_hbm, row_hbm, g_hbm, out_hbm,
          colv, rowv, xbuf, gbuf, acc, semx, semg):
        cid = lax.axis_index("c")
        sid = lax.axis_index("s")
        tile_chunk0 = cid * (_NS * n_chunks) + sid * n_chunks

        pltpu.sync_copy(col_hbm.at[pl.ds(tile_chunk0, n_chunks)], colv)
        pltpu.sync_copy(row_hbm.at[pl.ds(tile_chunk0, n_chunks)], rowv)

        def x_fire(g):
            pltpu.async_copy(x_hbm.at[colv.at[g]], xbuf, semx)

        def x_wait(g):
            pltpu.make_async_copy(x_hbm.at[colv.at[g]], xbuf, semx).wait()

        def g_fire(g):
            base = (tile_chunk0 + g) * _CH
            pltpu.async_copy(g_hbm.at[pl.ds(base, _CH)], gbuf, semg)

        def g_wait(g):
            base = (tile_chunk0 + g) * _CH
            pltpu.make_async_copy(g_hbm.at[pl.ds(base, _CH)], gbuf,
                                  semg).wait()

        @pl.loop(0, _CH)
        def _zero(i):
            for j in range(f // 16):
                xbuf[i, pl.ds(j * 16, 16)] = jnp.zeros((16,), jnp.float32)

        @pl.loop(0, rows_per_tile // _CH)
        def _blast(t):
            pltpu.sync_copy(
                xbuf,
                acc.at[pl.ds(sid * rows_per_tile + t * _CH, _CH)])

        plsc.subcore_barrier()

        x_fire(0)
        g_fire(0)

        @pl.loop(0, n_chunks)
        def _chunk(g):
            x_wait(g)
            g_wait(g)

            @pl.loop(0, _CH, unroll=4)
            def _mul(i):
                for j in range(f // 16):
                    sl = pl.ds(j * 16, 16)
                    gbuf[i, sl] = gbuf[i, sl] * xbuf[i, sl]

            @pl.when(g + 1 < n_chunks)
            def _fx():
                x_fire(g + 1)

            pltpu.sync_copy(gbuf, acc.at[rowv.at[g]], add=True)

            @pl.when(g + 1 < n_chunks)
            def _fg():
                g_fire(g + 1)

        plsc.subcore_barrier()

        pltpu.sync_copy(
            acc.at[pl.ds(sid * rows_per_tile, rows_per_tile)],
            out_hbm.at[cid, pl.ds(sid * rows_per_tile, rows_per_tile)])

    return k


def _lin_body(p_ref, w_ref, b_ref, o_ref):
    s = p_ref[0] + p_ref[1]
    dn = (((1,), (1,)), ((), ()))
    o_ref[...] = (lax.dot_general(s, w_ref[...], dn,
                                  preferred_element_type=jnp.float32)
                  + b_ref[...])


def _linear(partial, w, b2d, bn=1024):
    _, n, f = partial.shape
    out_dim = w.shape[0]
    grid = n // bn
    return pl.pallas_call(
        _lin_body,
        grid=(grid,),
        in_specs=[
            pl.BlockSpec((_NC, bn, f), lambda i: (0, i, 0)),
            pl.BlockSpec((out_dim, f), lambda i: (0, 0)),
            pl.BlockSpec((1, out_dim), lambda i: (0, 0)),
        ],
        out_specs=pl.BlockSpec((bn, out_dim), lambda i: (i, 0)),
        out_shape=jax.ShapeDtypeStruct((n, out_dim), jnp.float32),
    )(partial, w, b2d)


def kernel(x, edge_index, pseudo, mu, sigma, W, b):
    n, f = x.shape
    e = pseudo.shape[0]
    out_dim = W.shape[0]
    ei = edge_index.astype(jnp.int32)
    row, col = ei[0], ei[1]

    quant = _NW * _CH * 4
    epad = ((e + quant - 1) // quant) * quant
    pad = epad - e
    colp = jnp.pad(col, (0, pad))
    rowp = jnp.pad(row, (0, pad))
    pp = jnp.pad(pseudo, ((0, pad), (0, 0)))
    col2d = colp.reshape(epad // _CH, _CH)
    row2d = rowp.reshape(epad // _CH, _CH)

    npad = ((n + _NS * _CH - 1) // (_NS * _CH)) * (_NS * _CH)

    g = _gaussian(pp, mu.T, sigma.T, e)
    partial = _make_sc_scatter(npad, f, epad)(x, col2d, row2d, g)
    out = _linear(partial, W, b.reshape(1, out_dim))
    return out[:n]


# speedup vs baseline: 1.1960x; 1.1960x over previous
import functools

import jax
import jax.numpy as jnp
from jax import lax
from jax.experimental import pallas as pl
from jax.experimental.pallas import tpu as pltpu
from jax.experimental.pallas import tpu_sc as plsc

_NC = 2
_NS = 16
_NW = _NC * _NS
_CH = 64


def _gauss_body(e_total, p_ref, mut_ref, sgt_ref, g_ref):
    p = p_ref[...]
    a = -0.5 / (sgt_ref[...] ** 2 + 1e-14)
    mut = mut_ref[...]
    be, d = p.shape
    f = mut.shape[1]
    acc = jnp.zeros((be, f), jnp.float32)
    for dd in range(d):
        dif = p[:, dd:dd + 1] - mut[dd:dd + 1, :]
        acc = acc + a[dd:dd + 1, :] * (dif * dif)
    base = pl.program_id(0) * be
    eid = base + lax.broadcasted_iota(jnp.int32, (be, 1), 0)
    mask = (eid < e_total).astype(jnp.float32)
    g_ref[...] = jnp.exp(acc) * mask


def _gaussian(pseudo_pad, mut, sgt, e_total, be=2048):
    epad, d = pseudo_pad.shape
    f = mut.shape[1]
    grid = epad // be
    return pl.pallas_call(
        functools.partial(_gauss_body, e_total),
        grid=(grid,),
        in_specs=[
            pl.BlockSpec((be, d), lambda i: (i, 0)),
            pl.BlockSpec((d, f), lambda i: (0, 0)),
            pl.BlockSpec((d, f), lambda i: (0, 0)),
        ],
        out_specs=pl.BlockSpec((be, f), lambda i: (i, 0)),
        out_shape=jax.ShapeDtypeStruct((epad, f), jnp.float32),
    )(pseudo_pad, mut, sgt)


def _make_sc_scatter(n, f, epad):
    ew = epad // _NW
    n_chunks = ew // _CH
    rows_per_tile = n // _NS
    assert rows_per_tile * _NS == n and rows_per_tile % _CH == 0
    assert n_chunks % 4 == 0
    mesh = plsc.VectorSubcoreMesh(core_axis_name="c", subcore_axis_name="s",
                                  num_cores=_NC, num_subcores=_NS)

    @functools.partial(
        pl.kernel,
        out_type=jax.ShapeDtypeStruct((_NC, n, f), jnp.float32),
        mesh=mesh,
        scratch_types=[
            pltpu.VMEM((_CH,), jnp.int32),
            pltpu.VMEM((_CH,), jnp.int32),
            pltpu.VMEM((_CH,), jnp.int32),
            pltpu.VMEM((_CH,), jnp.int32),
            pltpu.VMEM((_CH,), jnp.int32),
            pltpu.VMEM((_CH,), jnp.int32),
            pltpu.VMEM((_CH,), jnp.int32),
            pltpu.VMEM((_CH,), jnp.int32),
            pltpu.VMEM((_CH, f), jnp.float32),
            pltpu.VMEM((_CH, f), jnp.float32),
            pltpu.VMEM((_CH, f), jnp.float32),
            pltpu.VMEM((_CH, f), jnp.float32),
            pltpu.VMEM_SHARED((n, f), jnp.float32),
            pltpu.SemaphoreType.DMA((4,)),
            pltpu.SemaphoreType.DMA((4,)),
            pltpu.SemaphoreType.DMA((2,)),
            pltpu.SemaphoreType.DMA((2,)),
        ],
    )
    def k(x_hbm, col_hbm, row_hbm, g_hbm, out_hbm,
          cv0, cv1, cv2, cv3, rv0, rv1, rv2, rv3,
          xb0, xb1, gb0, gb1, acc, semic, semir, semx, semg):
        cid = lax.axis_index("c")
        sid = lax.axis_index("s")
        tile_chunk0 = cid * (_NS * n_chunks) + sid * n_chunks
        colvs = (cv0, cv1, cv2, cv3)
        rowvs = (rv0, rv1, rv2, rv3)
        xbufs = (xb0, xb1)
        gbufs = (gb0, gb1)

        def idx_fire(s, g):
            pltpu.async_copy(col_hbm.at[tile_chunk0 + g], colvs[s],
                             semic.at[s])
            pltpu.async_copy(row_hbm.at[tile_chunk0 + g], rowvs[s],
                             semir.at[s])

        def idx_wait(s, g):
            pltpu.make_async_copy(col_hbm.at[tile_chunk0 + g], colvs[s],
                                  semic.at[s]).wait()
            pltpu.make_async_copy(row_hbm.at[tile_chunk0 + g], rowvs[s],
                                  semir.at[s]).wait()

        def data_fire(b, s, g):
            base = (tile_chunk0 + g) * _CH
            pltpu.async_copy(x_hbm.at[colvs[s]], xbufs[b], semx.at[b])
            pltpu.async_copy(g_hbm.at[pl.ds(base, _CH)], gbufs[b],
                             semg.at[b])

        def consume(b, s, g):
            base = (tile_chunk0 + g) * _CH
            pltpu.make_async_copy(
                x_hbm.at[colvs[s]], xbufs[b], semx.at[b]).wait()
            pltpu.make_async_copy(
                g_hbm.at[pl.ds(base, _CH)], gbufs[b], semg.at[b]).wait()
            xbuf, gbuf = xbufs[b], gbufs[b]

            @pl.loop(0, _CH, unroll=4)
            def _mul(i):
                for j in range(f // 16):
                    sl = pl.ds(j * 16, 16)
                    gbuf[i, sl] = gbuf[i, sl] * xbuf[i, sl]

            pltpu.sync_copy(gbuf, acc.at[rowvs[s]], add=True)

        idx_fire(0, 0)
        idx_fire(1, 1)

        @pl.loop(0, _CH)
        def _zero(i):
            for j in range(f // 16):
                xb0[i, pl.ds(j * 16, 16)] = jnp.zeros((16,), jnp.float32)

        @pl.loop(0, rows_per_tile // _CH)
        def _blast(t):
            pltpu.sync_copy(
                xb0,
                acc.at[pl.ds(sid * rows_per_tile + t * _CH, _CH)])

        plsc.subcore_barrier()

        idx_wait(0, 0)
        data_fire(0, 0, 0)

        @pl.loop(0, n_chunks // 4)
        def _quad(t):
            for kk in range(4):
                g = t * 4 + kk

                @pl.when(g + 2 < n_chunks)
                def _fi():
                    idx_fire((kk + 2) % 4, g + 2)

                @pl.when(g + 1 < n_chunks)
                def _fd():
                    idx_wait((kk + 1) % 4, g + 1)
                    data_fire((kk + 1) % 2, (kk + 1) % 4, g + 1)

                consume(kk % 2, kk, g)

        plsc.subcore_barrier()

        pltpu.sync_copy(
            acc.at[pl.ds(sid * rows_per_tile, rows_per_tile)],
            out_hbm.at[cid, pl.ds(sid * rows_per_tile, rows_per_tile)])

    return k


def _lin_body(pa_ref, pb_ref, w_ref, b_ref, o_ref):
    s = pa_ref[0] + pa_ref[1] + pb_ref[0] + pb_ref[1]
    dn = (((1,), (1,)), ((), ()))
    o_ref[...] = (lax.dot_general(s, w_ref[...], dn,
                                  preferred_element_type=jnp.float32)
                  + b_ref[...])


def _linear(pa, pb, w, b2d, bn=1024):
    _, n, f = pa.shape
    out_dim = w.shape[0]
    grid = n // bn
    return pl.pallas_call(
        _lin_body,
        grid=(grid,),
        in_specs=[
            pl.BlockSpec((_NC, bn, f), lambda i: (0, i, 0)),
            pl.BlockSpec((_NC, bn, f), lambda i: (0, i, 0)),
            pl.BlockSpec((out_dim, f), lambda i: (0, 0)),
            pl.BlockSpec((1, out_dim), lambda i: (0, 0)),
        ],
        out_specs=pl.BlockSpec((bn, out_dim), lambda i: (i, 0)),
        out_shape=jax.ShapeDtypeStruct((n, out_dim), jnp.float32),
    )(pa, pb, w, b2d)


def kernel(x, edge_index, pseudo, mu, sigma, W, b):
    n, f = x.shape
    e = pseudo.shape[0]
    out_dim = W.shape[0]
    ei = edge_index.astype(jnp.int32)
    row, col = ei[0], ei[1]

    quant = _NW * _CH * 4 * 2
    epad = ((e + quant - 1) // quant) * quant
    half = epad // 2
    pad = epad - e
    colp = jnp.pad(col, (0, pad))
    rowp = jnp.pad(row, (0, pad))
    pp = jnp.pad(pseudo, ((0, pad), (0, 0)))

    npad = ((n + _NS * _CH - 1) // (_NS * _CH)) * (_NS * _CH)
    sc = _make_sc_scatter(npad, f, half)
    mut, sgt = mu.T, sigma.T
    b2d = b.reshape(1, out_dim)

    def half_args(h):
        lo = h * half
        c2 = lax.dynamic_slice_in_dim(colp, lo, half).reshape(half // _CH, _CH)
        r2 = lax.dynamic_slice_in_dim(rowp, lo, half).reshape(half // _CH, _CH)
        p2 = lax.dynamic_slice_in_dim(pp, lo, half)
        return c2, r2, p2

    c2a, r2a, p2a = half_args(0)
    c2b, r2b, p2b = half_args(1)
    ga = _gaussian(p2a, mut, sgt, half)
    gb = _gaussian(p2b, mut, sgt, max(e - half, 0))
    pa = sc(x, c2a, r2a, ga)
    pb = sc(x, c2b, r2b, gb)
    out = _linear(pa, pb, W, b2d)
    return out[:n]


# speedup vs baseline: 1.3005x; 1.0874x over previous
import functools

import jax
import jax.numpy as jnp
from jax import lax
from jax.experimental import pallas as pl
from jax.experimental.pallas import tpu as pltpu
from jax.experimental.pallas import tpu_sc as plsc

_NC = 2
_NS = 16
_NW = _NC * _NS
_CH = 64


def _gauss_body(e_total, nblk, p_ref, mut_ref, a_ref, g_ref):
    p = p_ref[...]
    a = a_ref[...]
    mut = mut_ref[...]
    be, d = p.shape
    f = mut.shape[1]
    acc = jnp.zeros((be, f), jnp.float32)
    for dd in range(d):
        dif = p[:, dd:dd + 1] - mut[dd:dd + 1, :]
        acc = acc + a[dd:dd + 1, :] * (dif * dif)
    base = pl.program_id(0) * be
    eid = base + lax.broadcasted_iota(jnp.int32, (be, 1), 0)
    mask = (eid < e_total).astype(jnp.float32)
    g_ref[...] = jnp.exp(acc) * mask


def _gaussian(pseudo_pad, mut, atab, e_total, be=2048):
    epad, d = pseudo_pad.shape
    f = mut.shape[1]
    grid = epad // be
    return pl.pallas_call(
        functools.partial(_gauss_body, e_total, grid),
        grid=(grid,),
        in_specs=[
            pl.BlockSpec((be, d), lambda i: (i, 0)),
            pl.BlockSpec((d, f), lambda i: (0, 0)),
            pl.BlockSpec((d, f), lambda i: (0, 0)),
        ],
        out_specs=pl.BlockSpec((be, f), lambda i: (i, 0)),
        out_shape=jax.ShapeDtypeStruct((epad, f), jnp.float32),
    )(pseudo_pad, mut, atab)


def _make_sc_scatter(n, f, epad):
    ew = epad // _NW
    n_chunks = ew // _CH
    rows_per_tile = n // _NS
    assert rows_per_tile * _NS == n and rows_per_tile % _CH == 0
    assert n_chunks % 4 == 0
    mesh = plsc.VectorSubcoreMesh(core_axis_name="c", subcore_axis_name="s",
                                  num_cores=_NC, num_subcores=_NS)

    @functools.partial(
        pl.kernel,
        out_type=jax.ShapeDtypeStruct((_NC, n, f), jnp.float32),
        mesh=mesh,
        scratch_types=[
            pltpu.VMEM((_CH,), jnp.int32),
            pltpu.VMEM((_CH,), jnp.int32),
            pltpu.VMEM((_CH,), jnp.int32),
            pltpu.VMEM((_CH,), jnp.int32),
            pltpu.VMEM((_CH,), jnp.int32),
            pltpu.VMEM((_CH,), jnp.int32),
            pltpu.VMEM((_CH,), jnp.int32),
            pltpu.VMEM((_CH,), jnp.int32),
            pltpu.VMEM((_CH, f), jnp.float32),
            pltpu.VMEM((_CH, f), jnp.float32),
            pltpu.VMEM((_CH, f), jnp.float32),
            pltpu.VMEM((_CH, f), jnp.float32),
            pltpu.VMEM_SHARED((n, f), jnp.float32),
            pltpu.SemaphoreType.DMA((4,)),
            pltpu.SemaphoreType.DMA((4,)),
            pltpu.SemaphoreType.DMA((2,)),
            pltpu.SemaphoreType.DMA((2,)),
        ],
    )
    def k(x_hbm, col_hbm, row_hbm, g_hbm, out_hbm,
          cv0, cv1, cv2, cv3, rv0, rv1, rv2, rv3,
          xb0, xb1, gb0, gb1, acc, semic, semir, semx, semg):
        cid = lax.axis_index("c")
        sid = lax.axis_index("s")
        tile_chunk0 = cid * (_NS * n_chunks) + sid * n_chunks
        colvs = (cv0, cv1, cv2, cv3)
        rowvs = (rv0, rv1, rv2, rv3)
        xbufs = (xb0, xb1)
        gbufs = (gb0, gb1)

        def idx_fire(s, g):
            pltpu.async_copy(col_hbm.at[tile_chunk0 + g], colvs[s],
                             semic.at[s])
            pltpu.async_copy(row_hbm.at[tile_chunk0 + g], rowvs[s],
                             semir.at[s])

        def idx_wait(s, g):
            pltpu.make_async_copy(col_hbm.at[tile_chunk0 + g], colvs[s],
                                  semic.at[s]).wait()
            pltpu.make_async_copy(row_hbm.at[tile_chunk0 + g], rowvs[s],
                                  semir.at[s]).wait()

        def data_fire(b, s, g):
            base = (tile_chunk0 + g) * _CH
            pltpu.async_copy(x_hbm.at[colvs[s]], xbufs[b], semx.at[b])
            pltpu.async_copy(g_hbm.at[pl.ds(base, _CH)], gbufs[b],
                             semg.at[b])

        def consume(b, s, g):
            base = (tile_chunk0 + g) * _CH
            pltpu.make_async_copy(
                x_hbm.at[colvs[s]], xbufs[b], semx.at[b]).wait()
            pltpu.make_async_copy(
                g_hbm.at[pl.ds(base, _CH)], gbufs[b], semg.at[b]).wait()
            xbuf, gbuf = xbufs[b], gbufs[b]

            @pl.loop(0, _CH, unroll=4)
            def _mul(i):
                for j in range(f // 16):
                    sl = pl.ds(j * 16, 16)
                    gbuf[i, sl] = gbuf[i, sl] * xbuf[i, sl]

            pltpu.sync_copy(gbuf, acc.at[rowvs[s]], add=True)

        idx_fire(0, 0)
        idx_fire(1, 1)

        @pl.loop(0, _CH)
        def _zero(i):
            for j in range(f // 16):
                xb0[i, pl.ds(j * 16, 16)] = jnp.zeros((16,), jnp.float32)

        @pl.loop(0, rows_per_tile // _CH)
        def _blast(t):
            pltpu.sync_copy(
                xb0,
                acc.at[pl.ds(sid * rows_per_tile + t * _CH, _CH)])

        plsc.subcore_barrier()

        idx_wait(0, 0)
        data_fire(0, 0, 0)

        @pl.loop(0, n_chunks // 4)
        def _quad(t):
            for kk in range(4):
                g = t * 4 + kk

                @pl.when(g + 2 < n_chunks)
                def _fi():
                    idx_fire((kk + 2) % 4, g + 2)

                @pl.when(g + 1 < n_chunks)
                def _fd():
                    idx_wait((kk + 1) % 4, g + 1)
                    data_fire((kk + 1) % 2, (kk + 1) % 4, g + 1)

                consume(kk % 2, kk, g)

        plsc.subcore_barrier()

        pltpu.sync_copy(
            acc.at[pl.ds(sid * rows_per_tile, rows_per_tile)],
            out_hbm.at[cid, pl.ds(sid * rows_per_tile, rows_per_tile)])

    return k


def _lin_body(p_ref, w_ref, b_ref, o_ref):
    s = p_ref[0] + p_ref[1]
    dn = (((1,), (1,)), ((), ()))
    o_ref[...] = (lax.dot_general(s, w_ref[...], dn,
                                  preferred_element_type=jnp.float32)
                  + b_ref[...])


def _linear(partial, w, b2d, bn=1024):
    _, n, f = partial.shape
    out_dim = w.shape[0]
    grid = n // bn
    return pl.pallas_call(
        _lin_body,
        grid=(grid,),
        in_specs=[
            pl.BlockSpec((_NC, bn, f), lambda i: (0, i, 0)),
            pl.BlockSpec((out_dim, f), lambda i: (0, 0)),
            pl.BlockSpec((1, out_dim), lambda i: (0, 0)),
        ],
        out_specs=pl.BlockSpec((bn, out_dim), lambda i: (i, 0)),
        out_shape=jax.ShapeDtypeStruct((n, out_dim), jnp.float32),
    )(partial, w, b2d)


def kernel(x, edge_index, pseudo, mu, sigma, W, b):
    n, f = x.shape
    e = pseudo.shape[0]
    out_dim = W.shape[0]
    ei = edge_index.astype(jnp.int32)
    row, col = ei[0], ei[1]

    quant = _NW * _CH * 4
    epad = ((e + quant - 1) // quant) * quant
    pad = epad - e
    colp = jnp.pad(col, (0, pad))
    rowp = jnp.pad(row, (0, pad))
    pp = jnp.pad(pseudo, ((0, pad), (0, 0)))
    col2d = colp.reshape(epad // _CH, _CH)
    row2d = rowp.reshape(epad // _CH, _CH)

    npad = ((n + _NS * _CH - 1) // (_NS * _CH)) * (_NS * _CH)

    atab = -0.5 / (sigma.T ** 2 + 1e-14)
    g = _gaussian(pp, mu.T, atab, e)
    partial = _make_sc_scatter(npad, f, epad)(x, col2d, row2d, g)
    out = _linear(partial, W, b.reshape(1, out_dim))
    return out[:n]


# speedup vs baseline: 1.3440x; 1.0334x over previous
import functools

import jax
import jax.numpy as jnp
from jax import lax
from jax.experimental import pallas as pl
from jax.experimental.pallas import tpu as pltpu
from jax.experimental.pallas import tpu_sc as plsc

_NC = 2
_NS = 16
_NW = _NC * _NS
_CH = 64


def _gauss_body(e_total, nblk, p_ref, mut_ref, a_ref, g_ref):
    p = p_ref[...]
    a = a_ref[...]
    mut = mut_ref[...]
    be, d = p.shape
    f = mut.shape[1]
    acc = jnp.zeros((be, f), jnp.float32)
    for dd in range(d):
        dif = p[:, dd:dd + 1] - mut[dd:dd + 1, :]
        acc = acc + a[dd:dd + 1, :] * (dif * dif)
    base = pl.program_id(0) * be
    eid = base + lax.broadcasted_iota(jnp.int32, (be, 1), 0)
    mask = (eid < e_total).astype(jnp.float32)
    g_ref[...] = jnp.exp(acc) * mask


def _gaussian(pseudo_pad, mut, atab, e_total, be=4096):
    epad, d = pseudo_pad.shape
    f = mut.shape[1]
    grid = epad // be
    return pl.pallas_call(
        functools.partial(_gauss_body, e_total, grid),
        grid=(grid,),
        in_specs=[
            pl.BlockSpec((be, d), lambda i: (i, 0)),
            pl.BlockSpec((d, f), lambda i: (0, 0)),
            pl.BlockSpec((d, f), lambda i: (0, 0)),
        ],
        out_specs=pl.BlockSpec((be, f), lambda i: (i, 0)),
        out_shape=jax.ShapeDtypeStruct((epad, f), jnp.float32),
    )(pseudo_pad, mut, atab)


def _make_sc_scatter(n, f, epad):
    ew = epad // _NW
    n_chunks = ew // _CH
    rows_per_tile = n // _NS
    assert rows_per_tile * _NS == n and rows_per_tile % _CH == 0
    assert n_chunks % 4 == 0
    mesh = plsc.VectorSubcoreMesh(core_axis_name="c", subcore_axis_name="s",
                                  num_cores=_NC, num_subcores=_NS)

    @functools.partial(
        pl.kernel,
        out_type=jax.ShapeDtypeStruct((_NC, n, f), jnp.float32),
        mesh=mesh,
        scratch_types=[
            pltpu.VMEM((_CH,), jnp.int32),
            pltpu.VMEM((_CH,), jnp.int32),
            pltpu.VMEM((_CH,), jnp.int32),
            pltpu.VMEM((_CH,), jnp.int32),
            pltpu.VMEM((_CH,), jnp.int32),
            pltpu.VMEM((_CH,), jnp.int32),
            pltpu.VMEM((_CH,), jnp.int32),
            pltpu.VMEM((_CH,), jnp.int32),
            pltpu.VMEM((_CH, f), jnp.float32),
            pltpu.VMEM((_CH, f), jnp.float32),
            pltpu.VMEM((_CH, f), jnp.float32),
            pltpu.VMEM((_CH, f), jnp.float32),
            pltpu.VMEM_SHARED((n, f), jnp.float32),
            pltpu.SemaphoreType.DMA((4,)),
            pltpu.SemaphoreType.DMA((4,)),
            pltpu.SemaphoreType.DMA((2,)),
            pltpu.SemaphoreType.DMA((2,)),
        ],
    )
    def k(x_hbm, col_hbm, row_hbm, g_hbm, out_hbm,
          cv0, cv1, cv2, cv3, rv0, rv1, rv2, rv3,
          xb0, xb1, gb0, gb1, acc, semic, semir, semx, semg):
        cid = lax.axis_index("c")
        sid = lax.axis_index("s")
        tile_chunk0 = cid * (_NS * n_chunks) + sid * n_chunks
        colvs = (cv0, cv1, cv2, cv3)
        rowvs = (rv0, rv1, rv2, rv3)
        xbufs = (xb0, xb1)
        gbufs = (gb0, gb1)

        def idx_fire(s, g):
            pltpu.async_copy(col_hbm.at[tile_chunk0 + g], colvs[s],
                             semic.at[s])
            pltpu.async_copy(row_hbm.at[tile_chunk0 + g], rowvs[s],
                             semir.at[s])

        def idx_wait(s, g):
            pltpu.make_async_copy(col_hbm.at[tile_chunk0 + g], colvs[s],
                                  semic.at[s]).wait()
            pltpu.make_async_copy(row_hbm.at[tile_chunk0 + g], rowvs[s],
                                  semir.at[s]).wait()

        def data_fire(b, s, g):
            base = (tile_chunk0 + g) * _CH
            pltpu.async_copy(x_hbm.at[colvs[s]], xbufs[b], semx.at[b])
            pltpu.async_copy(g_hbm.at[pl.ds(base, _CH)], gbufs[b],
                             semg.at[b])

        def consume(b, s, g):
            base = (tile_chunk0 + g) * _CH
            pltpu.make_async_copy(
                x_hbm.at[colvs[s]], xbufs[b], semx.at[b]).wait()
            pltpu.make_async_copy(
                g_hbm.at[pl.ds(base, _CH)], gbufs[b], semg.at[b]).wait()
            xbuf, gbuf = xbufs[b], gbufs[b]

            @pl.loop(0, _CH, unroll=4)
            def _mul(i):
                for j in range(f // 16):
                    sl = pl.ds(j * 16, 16)
                    gbuf[i, sl] = gbuf[i, sl] * xbuf[i, sl]

            pltpu.sync_copy(gbuf, acc.at[rowvs[s]], add=True)

        idx_fire(0, 0)
        idx_fire(1, 1)

        @pl.loop(0, _CH)
        def _zero(i):
            for j in range(f // 16):
                xb0[i, pl.ds(j * 16, 16)] = jnp.zeros((16,), jnp.float32)

        @pl.loop(0, rows_per_tile // _CH)
        def _blast(t):
            pltpu.sync_copy(
                xb0,
                acc.at[pl.ds(sid * rows_per_tile + t * _CH, _CH)])

        plsc.subcore_barrier()

        idx_wait(0, 0)
        data_fire(0, 0, 0)

        @pl.loop(0, n_chunks // 4)
        def _quad(t):
            for kk in range(4):
                g = t * 4 + kk

                @pl.when(g + 2 < n_chunks)
                def _fi():
                    idx_fire((kk + 2) % 4, g + 2)

                @pl.when(g + 1 < n_chunks)
                def _fd():
                    idx_wait((kk + 1) % 4, g + 1)
                    data_fire((kk + 1) % 2, (kk + 1) % 4, g + 1)

                consume(kk % 2, kk, g)

        plsc.subcore_barrier()

        pltpu.sync_copy(
            acc.at[pl.ds(sid * rows_per_tile, rows_per_tile)],
            out_hbm.at[cid, pl.ds(sid * rows_per_tile, rows_per_tile)])

    return k


def _lin_body(p_ref, w_ref, b_ref, o_ref):
    s = p_ref[0] + p_ref[1]
    dn = (((1,), (1,)), ((), ()))
    o_ref[...] = (lax.dot_general(s, w_ref[...], dn,
                                  preferred_element_type=jnp.float32)
                  + b_ref[...])


def _linear(partial, w, b2d, bn=1024):
    _, n, f = partial.shape
    out_dim = w.shape[0]
    grid = n // bn
    return pl.pallas_call(
        _lin_body,
        grid=(grid,),
        in_specs=[
            pl.BlockSpec((_NC, bn, f), lambda i: (0, i, 0)),
            pl.BlockSpec((out_dim, f), lambda i: (0, 0)),
            pl.BlockSpec((1, out_dim), lambda i: (0, 0)),
        ],
        out_specs=pl.BlockSpec((bn, out_dim), lambda i: (i, 0)),
        out_shape=jax.ShapeDtypeStruct((n, out_dim), jnp.float32),
    )(partial, w, b2d)


def kernel(x, edge_index, pseudo, mu, sigma, W, b):
    n, f = x.shape
    e = pseudo.shape[0]
    out_dim = W.shape[0]
    ei = edge_index.astype(jnp.int32)
    row, col = ei[0], ei[1]

    quant = _NW * _CH * 4
    epad = ((e + quant - 1) // quant) * quant
    pad = epad - e
    colp = jnp.pad(col, (0, pad))
    rowp = jnp.pad(row, (0, pad))
    pp = jnp.pad(pseudo, ((0, pad), (0, 0)))
    col2d = colp.reshape(epad // _CH, _CH)
    row2d = rowp.reshape(epad // _CH, _CH)

    npad = ((n + _NS * _CH - 1) // (_NS * _CH)) * (_NS * _CH)

    atab = -0.5 / (sigma.T ** 2 + 1e-14)
    g = _gaussian(pp, mu.T, atab, e)
    partial = _make_sc_scatter(npad, f, epad)(x, col2d, row2d, g)
    out = _linear(partial, W, b.reshape(1, out_dim))
    return out[:n]


# speedup vs baseline: 1.4105x; 1.0495x over previous
import functools

import jax
import jax.numpy as jnp
from jax import lax
from jax.experimental import pallas as pl
from jax.experimental.pallas import tpu as pltpu
from jax.experimental.pallas import tpu_sc as plsc

_NC = 2
_NS = 16
_NW = _NC * _NS
_CH = 64


def _gauss_body(e_total, nblk, p_ref, mut_ref, a_ref, g_ref):
    p = p_ref[...]
    a = a_ref[...]
    mut = mut_ref[...]
    be, d = p.shape
    f = mut.shape[1]
    acc = jnp.zeros((be, f), jnp.float32)
    for dd in range(d):
        dif = p[:, dd:dd + 1] - mut[dd:dd + 1, :]
        acc = acc + a[dd:dd + 1, :] * (dif * dif)
    base = pl.program_id(0) * be
    eid = base + lax.broadcasted_iota(jnp.int32, (be, 1), 0)
    mask = (eid < e_total).astype(jnp.float32)
    g_ref[...] = jnp.exp(acc) * mask


def _gaussian(pseudo_pad, mut, atab, e_total, be=4096):
    epad, d = pseudo_pad.shape
    f = mut.shape[1]
    grid = epad // be
    return pl.pallas_call(
        functools.partial(_gauss_body, e_total, grid),
        grid=(grid,),
        in_specs=[
            pl.BlockSpec((be, d), lambda i: (i, 0)),
            pl.BlockSpec((d, f), lambda i: (0, 0)),
            pl.BlockSpec((d, f), lambda i: (0, 0)),
        ],
        out_specs=pl.BlockSpec((be, f), lambda i: (i, 0)),
        out_shape=jax.ShapeDtypeStruct((epad, f), jnp.float32),
    )(pseudo_pad, mut, atab)


def _make_sc_scatter(n, f, epad):
    ew = epad // _NW
    n_chunks = ew // _CH
    rows_per_tile = n // _NS
    assert rows_per_tile * _NS == n and rows_per_tile % _CH == 0
    assert n_chunks % 4 == 0
    mesh = plsc.VectorSubcoreMesh(core_axis_name="c", subcore_axis_name="s",
                                  num_cores=_NC, num_subcores=_NS)

    nc0 = ((n_chunks * 11) // 10) // 4 * 4
    nc1 = 2 * n_chunks - nc0
    assert nc1 % 4 == 0 and nc1 > 0

    @functools.partial(
        pl.kernel,
        out_type=jax.ShapeDtypeStruct((_NC, n, f), jnp.float32),
        mesh=mesh,
        scratch_types=[
            pltpu.VMEM((_CH,), jnp.int32),
            pltpu.VMEM((_CH,), jnp.int32),
            pltpu.VMEM((_CH,), jnp.int32),
            pltpu.VMEM((_CH,), jnp.int32),
            pltpu.VMEM((_CH,), jnp.int32),
            pltpu.VMEM((_CH,), jnp.int32),
            pltpu.VMEM((_CH,), jnp.int32),
            pltpu.VMEM((_CH,), jnp.int32),
            pltpu.VMEM((_CH, f), jnp.float32),
            pltpu.VMEM((_CH, f), jnp.float32),
            pltpu.VMEM((_CH, f), jnp.float32),
            pltpu.VMEM((_CH, f), jnp.float32),
            pltpu.VMEM_SHARED((n, f), jnp.float32),
            pltpu.SemaphoreType.DMA((4,)),
            pltpu.SemaphoreType.DMA((4,)),
            pltpu.SemaphoreType.DMA((2,)),
            pltpu.SemaphoreType.DMA((2,)),
        ],
    )
    def k(x_hbm, col_hbm, row_hbm, g_hbm, out_hbm,
          cv0, cv1, cv2, cv3, rv0, rv1, rv2, rv3,
          xb0, xb1, gb0, gb1, acc, semic, semir, semx, semg):
        cid = lax.axis_index("c")
        sid = lax.axis_index("s")
        on0 = cid == 0
        nloc = jnp.where(on0, nc0, nc1)
        tile_chunk0 = jnp.where(on0, sid * nc0, _NS * nc0 + sid * nc1)
        colvs = (cv0, cv1, cv2, cv3)
        rowvs = (rv0, rv1, rv2, rv3)
        xbufs = (xb0, xb1)
        gbufs = (gb0, gb1)

        def idx_fire(s, g):
            pltpu.async_copy(col_hbm.at[tile_chunk0 + g], colvs[s],
                             semic.at[s])
            pltpu.async_copy(row_hbm.at[tile_chunk0 + g], rowvs[s],
                             semir.at[s])

        def idx_wait(s, g):
            pltpu.make_async_copy(col_hbm.at[tile_chunk0 + g], colvs[s],
                                  semic.at[s]).wait()
            pltpu.make_async_copy(row_hbm.at[tile_chunk0 + g], rowvs[s],
                                  semir.at[s]).wait()

        def data_fire(b, s, g):
            base = (tile_chunk0 + g) * _CH
            pltpu.async_copy(x_hbm.at[colvs[s]], xbufs[b], semx.at[b])
            pltpu.async_copy(g_hbm.at[pl.ds(base, _CH)], gbufs[b],
                             semg.at[b])

        def consume(b, s, g):
            base = (tile_chunk0 + g) * _CH
            pltpu.make_async_copy(
                x_hbm.at[colvs[s]], xbufs[b], semx.at[b]).wait()
            pltpu.make_async_copy(
                g_hbm.at[pl.ds(base, _CH)], gbufs[b], semg.at[b]).wait()
            xbuf, gbuf = xbufs[b], gbufs[b]

            @pl.loop(0, _CH, unroll=4)
            def _mul(i):
                for j in range(f // 16):
                    sl = pl.ds(j * 16, 16)
                    gbuf[i, sl] = gbuf[i, sl] * xbuf[i, sl]

            pltpu.sync_copy(gbuf, acc.at[rowvs[s]], add=True)

        idx_fire(0, 0)
        idx_fire(1, 1)

        @pl.loop(0, _CH)
        def _zero(i):
            for j in range(f // 16):
                xb0[i, pl.ds(j * 16, 16)] = jnp.zeros((16,), jnp.float32)

        @pl.loop(0, rows_per_tile // _CH)
        def _blast(t):
            pltpu.sync_copy(
                xb0,
                acc.at[pl.ds(sid * rows_per_tile + t * _CH, _CH)])

        plsc.subcore_barrier()

        idx_wait(0, 0)
        data_fire(0, 0, 0)

        @pl.loop(0, nloc // 4)
        def _quad(t):
            for kk in range(4):
                g = t * 4 + kk

                @pl.when(g + 2 < nloc)
                def _fi():
                    idx_fire((kk + 2) % 4, g + 2)

                @pl.when(g + 1 < nloc)
                def _fd():
                    idx_wait((kk + 1) % 4, g + 1)
                    data_fire((kk + 1) % 2, (kk + 1) % 4, g + 1)

                consume(kk % 2, kk, g)

        plsc.subcore_barrier()

        pltpu.sync_copy(
            acc.at[pl.ds(sid * rows_per_tile, rows_per_tile)],
            out_hbm.at[cid, pl.ds(sid * rows_per_tile, rows_per_tile)])

    return k


def _lin_body(p_ref, w_ref, b_ref, o_ref):
    s = p_ref[0] + p_ref[1]
    dn = (((1,), (1,)), ((), ()))
    o_ref[...] = (lax.dot_general(s, w_ref[...], dn,
                                  preferred_element_type=jnp.float32)
                  + b_ref[...])


def _linear(partial, w, b2d, bn=1024):
    _, n, f = partial.shape
    out_dim = w.shape[0]
    grid = n // bn
    return pl.pallas_call(
        _lin_body,
        grid=(grid,),
        in_specs=[
            pl.BlockSpec((_NC, bn, f), lambda i: (0, i, 0)),
            pl.BlockSpec((out_dim, f), lambda i: (0, 0)),
            pl.BlockSpec((1, out_dim), lambda i: (0, 0)),
        ],
        out_specs=pl.BlockSpec((bn, out_dim), lambda i: (i, 0)),
        out_shape=jax.ShapeDtypeStruct((n, out_dim), jnp.float32),
    )(partial, w, b2d)


def kernel(x, edge_index, pseudo, mu, sigma, W, b):
    n, f = x.shape
    e = pseudo.shape[0]
    out_dim = W.shape[0]
    ei = edge_index.astype(jnp.int32)
    row, col = ei[0], ei[1]

    quant = _NW * _CH * 4
    epad = ((e + quant - 1) // quant) * quant
    pad = epad - e
    colp = jnp.pad(col, (0, pad))
    rowp = jnp.pad(row, (0, pad))
    pp = jnp.pad(pseudo, ((0, pad), (0, 0)))
    col2d = colp.reshape(epad // _CH, _CH)
    row2d = rowp.reshape(epad // _CH, _CH)

    npad = ((n + _NS * _CH - 1) // (_NS * _CH)) * (_NS * _CH)

    atab = -0.5 / (sigma.T ** 2 + 1e-14)
    g = _gaussian(pp, mu.T, atab, e)
    partial = _make_sc_scatter(npad, f, epad)(x, col2d, row2d, g)
    out = _linear(partial, W, b.reshape(1, out_dim))
    return out[:n]
